# baseline (speedup 1.0000x reference)
import functools

import jax
import jax.numpy as jnp
from jax import lax
from jax.experimental import pallas as pl
from jax.experimental.pallas import tpu as pltpu
from jax.experimental.pallas import tpu_sc as plsc

H = 768
LANES = 16
VPR = H // LANES
CHUNK = 32
NC, NS = 2, 16
NW = NC * NS
EPS = 1e-12


def _rsqrt(x):
    i = plsc.bitcast(x, jnp.int32)
    i = jnp.int32(0x5F3759DF) - lax.shift_right_logical(i, 1)
    y = plsc.bitcast(i, jnp.float32)
    for _ in range(3):
        y = y * (1.5 - 0.5 * x * y * y)
    return y


def _hsum(x):
    lanes = lax.iota(jnp.int32, LANES)
    dnums = lax.GatherDimensionNumbers(
        offset_dims=(), collapsed_slice_dims=(0,), start_index_map=(0,))
    for sh in (8, 4, 2, 1):
        perm = (lanes ^ sh)[:, None]
        x = x + lax.gather(x, perm, dnums, slice_sizes=(1,),
                           mode=lax.GatherScatterMode.PROMISE_IN_BOUNDS)
    return x


def _emb_body(ids_hbm, word_hbm, pos_hbm, typ_hbm, gam_hbm, bet_hbm, out_hbm,
              idx_v, w_v, p_v, typ_v, gam_v, bet_v, gsem, psem):
    n_rows = out_hbm.shape[0]
    seq_len = pos_hbm.shape[0]
    rows_per_tile = n_rows // NW
    nchunk = rows_per_tile // CHUNK

    wid = lax.axis_index("s") * NC + lax.axis_index("c")
    base = wid * rows_per_tile

    pltpu.sync_copy(ids_hbm.at[wid], idx_v)
    pltpu.sync_copy(typ_hbm, typ_v)
    pltpu.sync_copy(gam_hbm, gam_v)
    pltpu.sync_copy(bet_hbm, bet_v)

    def chunk_body(c, _):
        s0 = base + c * CHUNK
        pos0 = lax.rem(s0, seq_len)
        g = pltpu.async_copy(word_hbm.at[idx_v.at[c]], w_v, gsem)
        p = pltpu.async_copy(pos_hbm.at[pl.ds(pos0, CHUNK)], p_v, psem)
        g.wait()
        p.wait()

        def row_body(r, _):
            def pass1(j, carry):
                acc, accsq = carry
                sl = pl.ds(j * LANES, LANES)
                e = w_v[r, sl] + p_v[r, sl] + typ_v[sl]
                w_v[r, sl] = e
                return acc + e, accsq + e * e

            zero = jnp.zeros((LANES,), jnp.float32)
            acc, accsq = lax.fori_loop(0, VPR, pass1, (zero, zero))
            tot = _hsum(acc)
            tot2 = _hsum(accsq)
            mean = tot * (1.0 / H)
            var = tot2 * (1.0 / H) - mean * mean
            rstd = _rsqrt(var + EPS)

            def pass2(j, _):
                sl = pl.ds(j * LANES, LANES)
                e = w_v[r, sl]
                w_v[r, sl] = (e - mean) * rstd * gam_v[sl] + bet_v[sl]
                return 0

            lax.fori_loop(0, VPR, pass2, 0)
            return 0

        lax.fori_loop(0, CHUNK, row_body, 0)
        pltpu.sync_copy(w_v, out_hbm.at[pl.ds(s0, CHUNK)])
        return 0

    lax.fori_loop(0, nchunk, chunk_body, 0)


def kernel(input_ids, extended_attention_mask, word_table, pos_table,
           type_table, gamma, beta):
    b, s = input_ids.shape
    n_rows = b * s
    ids3 = input_ids.reshape(NW, n_rows // (NW * CHUNK), CHUNK)
    type_row = type_table[0]

    mesh = plsc.VectorSubcoreMesh(core_axis_name="c", subcore_axis_name="s")
    run = functools.partial(
        pl.kernel,
        out_type=jax.ShapeDtypeStruct((n_rows, H), jnp.float32),
        mesh=mesh,
        compiler_params=pltpu.CompilerParams(needs_layout_passes=False),
        scratch_types=[
            pltpu.VMEM(ids3.shape[1:], jnp.int32),
            pltpu.VMEM((CHUNK, H), jnp.float32),
            pltpu.VMEM((CHUNK, H), jnp.float32),
            pltpu.VMEM((H,), jnp.float32),
            pltpu.VMEM((H,), jnp.float32),
            pltpu.VMEM((H,), jnp.float32),
            pltpu.SemaphoreType.DMA,
            pltpu.SemaphoreType.DMA,
        ],
    )(_emb_body)
    emb = run(ids3, word_table, pos_table, type_row, gamma, beta)
    return emb.reshape(b, s, H), extended_attention_mask
